# baseline (speedup 1.0000x reference)
import functools

import jax
import jax.numpy as jnp
from jax.experimental import pallas as pl
from jax.experimental.pallas import tpu as pltpu

_NEG = 0.5
_BM0 = 400
_GRP = 5
_BM1 = 2000


def _leaky(x):
    return jnp.where(x >= 0, x, _NEG * x)


def _fused(a_ref, e_ref, o_ref, a_sc, e_sc, h_sc, hb_sc, *, nb0):
    i = pl.program_id(0)

    @pl.when(i < nb0)
    def _():
        a_sc[pl.ds(i * _BM0, _BM0), :] = a_ref[...].astype(jnp.bfloat16)
        e_sc[pl.ds(i * _BM0, _BM0), :] = e_ref[...].astype(jnp.bfloat16)

        @pl.when(i % _GRP == _GRP - 1)
        def _():
            g = i // _GRP
            rows = _GRP * _BM0
            part = jax.lax.dot_general(
                a_sc[pl.ds(g * rows, rows), :],
                e_sc[pl.ds(g * rows, rows), :],
                (((0,), (0,)), ((), ())),
                preferred_element_type=jnp.float32)

            @pl.when(g == 0)
            def _():
                h_sc[...] = part

            @pl.when(g > 0)
            def _():
                h_sc[...] += part

        @pl.when(i == nb0 - 1)
        def _():
            hb_sc[...] = _leaky(h_sc[...]).astype(jnp.bfloat16)

    @pl.when(i >= nb0)
    def _():
        j = i - nb0
        o_ref[...] = _leaky(jnp.dot(a_sc[pl.ds(j * _BM1, _BM1), :],
                                    hb_sc[...],
                                    preferred_element_type=jnp.float32))


def kernel(adj, embeds):
    n, e = adj.shape
    d = embeds.shape[1]
    nb0 = n // _BM0
    nb1 = n // _BM1
    body = functools.partial(_fused, nb0=nb0)
    return pl.pallas_call(
        body,
        grid=(nb0 + nb1,),
        in_specs=[
            pl.BlockSpec((_BM0, e), lambda i: (jnp.minimum(i, nb0 - 1), 0)),
            pl.BlockSpec((_BM0, d), lambda i: (jnp.minimum(i, nb0 - 1), 0)),
        ],
        out_specs=pl.BlockSpec((_BM1, d), lambda i: (jnp.maximum(i - nb0, 0), 0)),
        out_shape=jax.ShapeDtypeStruct((n, d), jnp.float32),
        scratch_shapes=[
            pltpu.VMEM((n, e), jnp.bfloat16),
            pltpu.VMEM((n, d), jnp.bfloat16),
            pltpu.VMEM((e, d), jnp.float32),
            pltpu.VMEM((e, d), jnp.bfloat16),
        ],
    )(adj, embeds)


# speedup vs baseline: 1.4937x; 1.4937x over previous
import jax
import jax.numpy as jnp
from jax.experimental import pallas as pl
from jax.experimental.pallas import tpu as pltpu

_NEG = 0.5
_BE = 256


def _leaky(x):
    return jnp.where(x >= 0, x, _NEG * x)


def _body(a_ref, e_ref, o_ref, e_sc):
    k = pl.program_id(0)
    ne = pl.num_programs(0)

    @pl.when(k == 0)
    def _():
        e_sc[...] = e_ref[...].astype(jnp.bfloat16)
        o_ref[...] = jnp.zeros_like(o_ref)

    ab = a_ref[...].astype(jnp.bfloat16)
    hk = _leaky(jax.lax.dot_general(
        ab, e_sc[...], (((0,), (0,)), ((), ())),
        preferred_element_type=jnp.float32)).astype(jnp.bfloat16)
    o_ref[...] += jax.lax.dot_general(
        ab, hk, (((1,), (0,)), ((), ())),
        preferred_element_type=jnp.float32)

    @pl.when(k == ne - 1)
    def _():
        o_ref[...] = _leaky(o_ref[...])


def kernel(adj, embeds):
    n, e = adj.shape
    d = embeds.shape[1]
    ne = e // _BE
    return pl.pallas_call(
        _body,
        grid=(ne,),
        in_specs=[
            pl.BlockSpec((n, _BE), lambda k: (0, k)),
            pl.BlockSpec((n, d), lambda k: (0, 0)),
        ],
        out_specs=pl.BlockSpec((n, d), lambda k: (0, 0)),
        out_shape=jax.ShapeDtypeStruct((n, d), jnp.float32),
        scratch_shapes=[
            pltpu.VMEM((n, d), jnp.bfloat16),
        ],
    )(adj, embeds)
